# baseline (speedup 1.0000x reference)
import jax
import jax.numpy as jnp
from jax import lax
from jax.experimental import pallas as pl
from jax.experimental.pallas import tpu as pltpu
from jax.experimental.pallas import tpu_sc as plsc

B = 128
P = 512 * 512
NSEG = 9
L = 16
NC = 2
NS = 16
NW = NC * NS
CHUNK = 1792
NBUF = 4
RG = 8
NRG = B // RG
SEGW = NSEG * L
ACC = B * SEGW
COUT = 4
NCLS = 8
PSPLIT = 114688
PW = PSPLIT // NW
NPC = PW // CHUNK
NSTEPS = NPC * NRG
BLKC = 8192
TCOFF = PSPLIT // BLKC
TCBLKS = (P - PSPLIT) // BLKC


def _sc_body(x_hbm, seg_hbm, out_hbm, segv, idxp, acc2,
             xb0, xb1, sem0, sem1, fsem):
    cid = lax.axis_index("c")
    sid = lax.axis_index("s")
    wid = sid * NC + cid
    wbase = wid * PW

    pltpu.sync_copy(seg_hbm.at[pl.ds(wbase, PW)], segv)
    lane = lax.iota(jnp.int32, L)

    def ib(i, c):
        idxp[pl.ds(i * L, L)] = segv[pl.ds(i * L, L)] * L + lane
        return c

    lax.fori_loop(0, PW // L, ib, 0)

    zv = jnp.zeros((L,), jnp.float32)

    def zb(n, c):
        for j in range(SEGW // L):
            acc2[n, pl.ds(j * L, L)] = zv
        return c

    lax.fori_loop(0, B, zb, 0)

    bufs = (xb0, xb1)
    sems = (sem0, sem1)

    def start(rg, b):
        pltpu.async_copy(
            x_hbm.at[pl.ds(rg * RG, RG), pl.ds(wbase, PW)],
            bufs[b], sems[b])

    def wait(b):
        pltpu.make_async_copy(
            x_hbm.at[pl.ds(0, RG), pl.ds(0, PW)], bufs[b], sems[b]).wait()

    def compute(rg, b):
        buf = bufs[b]
        rows = [jnp.full((L,), rg * RG + r, jnp.int32) for r in range(RG)]

        @plsc.parallel_loop(0, PW // L, unroll=2)
        def vb(v):
            iv = idxp[pl.ds(v * L, L)]
            xs = [buf[r, pl.ds(v * L, L)] for r in range(RG)]
            for r in range(RG):
                plsc.addupdate_scatter(acc2, [rows[r], iv], xs[r])

    def flush(rg):
        pltpu.async_copy(acc2.at[pl.ds(rg * RG, RG), :],
                         out_hbm.at[wid, pl.ds(rg * RG, RG), :], fsem)

    start(0, 0)
    start(1, 1)

    def pair(tp, c):
        for b in range(2):
            t = tp * 2 + b
            wait(b)
            compute(t, b)
            start(t + 2, b)
            flush(t)
        return c

    lax.fori_loop(0, NRG // 2 - 1, pair, 0)
    for b in range(2):
        wait(b)
        compute(NRG - 2 + b, b)
        flush(NRG - 2 + b)
    for _ in range(NRG):
        pltpu.make_async_copy(acc2.at[pl.ds(0, RG), :],
                              out_hbm.at[wid, pl.ds(0, RG), :], fsem).wait()


_sc_call = pl.kernel(
    _sc_body,
    out_type=jax.ShapeDtypeStruct((NW, B, SEGW), jnp.float32),
    mesh=plsc.VectorSubcoreMesh(
        core_axis_name="c", subcore_axis_name="s",
        num_cores=NC, num_subcores=NS),
    compiler_params=pltpu.CompilerParams(needs_layout_passes=False),
    scratch_types=[
        pltpu.VMEM((PW,), jnp.int32),
        pltpu.VMEM((PW,), jnp.int32),
        pltpu.VMEM((B, SEGW), jnp.float32),
        pltpu.VMEM((RG, PW), jnp.float32),
        pltpu.VMEM((RG, PW), jnp.float32),
        pltpu.SemaphoreType.DMA,
        pltpu.SemaphoreType.DMA,
        pltpu.SemaphoreType.DMA,
    ],
)


def _tc_seg_body(x_ref, seg_ref, out_ref):
    i = pl.program_id(0)
    sb = seg_ref[0]
    oh = (sb == lax.broadcasted_iota(jnp.int32, (L, BLKC), 0)
          ).astype(jnp.float32)
    part = lax.dot_general(x_ref[...], oh, (((1,), (1,)), ((), ())),
                           preferred_element_type=jnp.float32)

    @pl.when(i == 0)
    def _():
        out_ref[...] = part

    @pl.when(i > 0)
    def _():
        out_ref[...] += part


_tc_seg_call = pl.pallas_call(
    _tc_seg_body,
    grid=(TCBLKS,),
    in_specs=[
        pl.BlockSpec((B, BLKC), lambda i: (0, TCOFF + i)),
        pl.BlockSpec((1, 1, BLKC), lambda i: (TCOFF + i, 0, 0)),
    ],
    out_specs=pl.BlockSpec((B, L), lambda i: (0, 0)),
    out_shape=jax.ShapeDtypeStruct((B, L), jnp.float32),
)


NFGL = NSEG * COUT


def _ep_body(p_ref, tcp_ref, wpack_ref, wfc_ref, out_ref, acc_sc):
    w = pl.program_id(0)

    @pl.when(w == 0)
    def _():
        acc_sc[...] = p_ref[0]

    @pl.when(w > 0)
    def _():
        acc_sc[...] += p_ref[0]

    @pl.when(w == NW - 1)
    def _():
        sel = (lax.broadcasted_iota(jnp.int32, (SEGW, L), 0) // L ==
               lax.broadcasted_iota(jnp.int32, (SEGW, L), 1)).astype(jnp.float32)
        seg16 = (jnp.dot(acc_sc[...], sel, preferred_element_type=jnp.float32)
                 + tcp_ref[...])
        rep = (lax.broadcasted_iota(jnp.int32, (L, NFGL), 1) // COUT ==
               lax.broadcasted_iota(jnp.int32, (L, NFGL), 0)).astype(jnp.float32)
        seg_rep = jnp.dot(seg16, rep, preferred_element_type=jnp.float32)
        wf = jnp.reshape(wpack_ref[pl.ds(0, NFGL)], (1, NFGL))
        bf = jnp.reshape(wpack_ref[pl.ds(NFGL, NFGL)], (1, NFGL))
        bc = jnp.reshape(wpack_ref[pl.ds(2 * NFGL, NCLS)], (1, NCLS))
        flat = seg_rep * wf + bf
        out_ref[...] = (jnp.dot(flat, wfc_ref[...],
                                preferred_element_type=jnp.float32) + bc)


_ep_call = pl.pallas_call(
    _ep_body,
    grid=(NW,),
    in_specs=[
        pl.BlockSpec((1, B, SEGW), lambda w: (w, 0, 0)),
        pl.BlockSpec((B, L), lambda w: (0, 0)),
        pl.BlockSpec((2 * NFGL + NCLS,), lambda w: (0,)),
        pl.BlockSpec((NFGL, NCLS), lambda w: (0, 0)),
    ],
    out_specs=pl.BlockSpec((B, NCLS), lambda w: (0, 0)),
    out_shape=jax.ShapeDtypeStruct((B, NCLS), jnp.float32),
    scratch_shapes=[pltpu.VMEM((B, SEGW), jnp.float32)],
)


def kernel(x, segment_ids, W_fgl, b_fgl, W_fc, b_fc):
    seg32 = segment_ids.astype(jnp.int32)
    part = _sc_call(x, seg32)
    tcp = _tc_seg_call(x, seg32.reshape(P // BLKC, 1, BLKC))
    wpack = jnp.concatenate([W_fgl.reshape(-1), b_fgl.reshape(-1), b_fc])
    return _ep_call(part, tcp, wpack, W_fc)


# speedup vs baseline: 1.2221x; 1.2221x over previous
import jax
import jax.numpy as jnp
from jax import lax
from jax.experimental import pallas as pl
from jax.experimental.pallas import tpu as pltpu
from jax.experimental.pallas import tpu_sc as plsc

B = 128
P = 512 * 512
NSEG = 9
L = 16
NC = 2
NS = 16
NW = NC * NS
CHUNK = 1792
NBUF = 4
RG = 8
NRG = B // RG
SEGW = NSEG * L
ACC = B * SEGW
COUT = 4
NCLS = 8
PSPLIT = 106496
PW = PSPLIT // NW
NPC = PW // CHUNK
NSTEPS = NPC * NRG
BLKC = 8192
TCOFF = PSPLIT // BLKC
TCBLKS = (P - PSPLIT) // BLKC


def _sc_body(x_hbm, seg_hbm, out_hbm, segv, idxp, acc2,
             xb0, xb1, sem0, sem1, fsem):
    cid = lax.axis_index("c")
    sid = lax.axis_index("s")
    wid = sid * NC + cid
    wbase = wid * PW

    pltpu.sync_copy(seg_hbm.at[pl.ds(wbase, PW)], segv)
    lane = lax.iota(jnp.int32, L)

    def ib(i, c):
        idxp[pl.ds(i * L, L)] = segv[pl.ds(i * L, L)] * L + lane
        return c

    lax.fori_loop(0, PW // L, ib, 0)

    zv = jnp.zeros((L,), jnp.float32)

    def zb(n, c):
        for j in range(SEGW // L):
            acc2[n, pl.ds(j * L, L)] = zv
        return c

    lax.fori_loop(0, B, zb, 0)

    bufs = (xb0, xb1)
    sems = (sem0, sem1)

    def start(rg, b):
        pltpu.async_copy(
            x_hbm.at[pl.ds(rg * RG, RG), pl.ds(wbase, PW)],
            bufs[b], sems[b])

    def wait(b):
        pltpu.make_async_copy(
            x_hbm.at[pl.ds(0, RG), pl.ds(0, PW)], bufs[b], sems[b]).wait()

    def compute(rg, b):
        buf = bufs[b]
        rows = [jnp.full((L,), rg * RG + r, jnp.int32) for r in range(RG)]

        @plsc.parallel_loop(0, PW // L, unroll=2)
        def vb(v):
            iv = idxp[pl.ds(v * L, L)]
            xs = [buf[r, pl.ds(v * L, L)] for r in range(RG)]
            for r in range(RG):
                plsc.addupdate_scatter(acc2, [rows[r], iv], xs[r])

    def flush(rg):
        pltpu.async_copy(acc2.at[pl.ds(rg * RG, RG), :],
                         out_hbm.at[wid, pl.ds(rg * RG, RG), :], fsem)

    start(0, 0)
    start(1, 1)

    def pair(tp, c):
        for b in range(2):
            t = tp * 2 + b
            wait(b)
            compute(t, b)
            start(t + 2, b)
            flush(t)
        return c

    lax.fori_loop(0, NRG // 2 - 1, pair, 0)
    for b in range(2):
        wait(b)
        compute(NRG - 2 + b, b)
        flush(NRG - 2 + b)
    for _ in range(NRG):
        pltpu.make_async_copy(acc2.at[pl.ds(0, RG), :],
                              out_hbm.at[wid, pl.ds(0, RG), :], fsem).wait()


_sc_call = pl.kernel(
    _sc_body,
    out_type=jax.ShapeDtypeStruct((NW, B, SEGW), jnp.float32),
    mesh=plsc.VectorSubcoreMesh(
        core_axis_name="c", subcore_axis_name="s",
        num_cores=NC, num_subcores=NS),
    compiler_params=pltpu.CompilerParams(needs_layout_passes=False),
    scratch_types=[
        pltpu.VMEM((PW,), jnp.int32),
        pltpu.VMEM((PW,), jnp.int32),
        pltpu.VMEM((B, SEGW), jnp.float32),
        pltpu.VMEM((RG, PW), jnp.float32),
        pltpu.VMEM((RG, PW), jnp.float32),
        pltpu.SemaphoreType.DMA,
        pltpu.SemaphoreType.DMA,
        pltpu.SemaphoreType.DMA,
    ],
)


def _tc_seg_body(x_ref, seg_ref, out_ref):
    i = pl.program_id(0)
    sb = seg_ref[0]
    oh = (sb == lax.broadcasted_iota(jnp.int32, (L, BLKC), 0)
          ).astype(jnp.float32)
    part = lax.dot_general(x_ref[...], oh, (((1,), (1,)), ((), ())),
                           preferred_element_type=jnp.float32)

    @pl.when(i == 0)
    def _():
        out_ref[...] = part

    @pl.when(i > 0)
    def _():
        out_ref[...] += part


_tc_seg_call = pl.pallas_call(
    _tc_seg_body,
    grid=(TCBLKS,),
    in_specs=[
        pl.BlockSpec((B, BLKC), lambda i: (0, TCOFF + i)),
        pl.BlockSpec((1, 1, BLKC), lambda i: (TCOFF + i, 0, 0)),
    ],
    out_specs=pl.BlockSpec((B, L), lambda i: (0, 0)),
    out_shape=jax.ShapeDtypeStruct((B, L), jnp.float32),
)


NFGL = NSEG * COUT


def _ep_body(p_ref, tcp_ref, wpack_ref, wfc_ref, out_ref):
    acc = p_ref[0]
    for w in range(1, NW):
        acc = acc + p_ref[w]
    sel = (lax.broadcasted_iota(jnp.int32, (SEGW, L), 0) // L ==
           lax.broadcasted_iota(jnp.int32, (SEGW, L), 1)).astype(jnp.float32)
    seg16 = (jnp.dot(acc, sel, preferred_element_type=jnp.float32)
             + tcp_ref[...])
    rep = (lax.broadcasted_iota(jnp.int32, (L, NFGL), 1) // COUT ==
           lax.broadcasted_iota(jnp.int32, (L, NFGL), 0)).astype(jnp.float32)
    seg_rep = jnp.dot(seg16, rep, preferred_element_type=jnp.float32)
    wf = jnp.reshape(wpack_ref[pl.ds(0, NFGL)], (1, NFGL))
    bf = jnp.reshape(wpack_ref[pl.ds(NFGL, NFGL)], (1, NFGL))
    bc = jnp.reshape(wpack_ref[pl.ds(2 * NFGL, NCLS)], (1, NCLS))
    flat = seg_rep * wf + bf
    out_ref[...] = (jnp.dot(flat, wfc_ref[...],
                            preferred_element_type=jnp.float32) + bc)


_ep_call = pl.pallas_call(
    _ep_body,
    out_shape=jax.ShapeDtypeStruct((B, NCLS), jnp.float32),
)


def kernel(x, segment_ids, W_fgl, b_fgl, W_fc, b_fc):
    seg32 = segment_ids.astype(jnp.int32)
    part = _sc_call(x, seg32)
    tcp = _tc_seg_call(x, seg32.reshape(P // BLKC, 1, BLKC))
    wpack = jnp.concatenate([W_fgl.reshape(-1), b_fgl.reshape(-1), b_fc])
    return _ep_call(part, tcp, wpack, W_fc)


# speedup vs baseline: 1.2367x; 1.0119x over previous
import jax
import jax.numpy as jnp
from jax import lax
from jax.experimental import pallas as pl
from jax.experimental.pallas import tpu as pltpu
from jax.experimental.pallas import tpu_sc as plsc

B = 128
P = 512 * 512
NSEG = 9
L = 16
NC = 2
NS = 16
NW = NC * NS
CHUNK = 1792
NBUF = 4
RG = 8
NRG = B // RG
SEGW = NSEG * L
ACC = B * SEGW
COUT = 4
NCLS = 8
PSPLIT = 106496
PW = PSPLIT // NW
NPC = PW // CHUNK
NSTEPS = NPC * NRG
BLKC = 8192
TCOFF = PSPLIT // BLKC
TCBLKS = (P - PSPLIT) // BLKC


def _sc_body(x_hbm, seg_hbm, out_hbm, segv, idxp, acc2,
             xb0, xb1, sem0, sem1, fsem):
    cid = lax.axis_index("c")
    sid = lax.axis_index("s")
    wid = sid * NC + cid
    wbase = wid * PW

    pltpu.async_copy(x_hbm.at[pl.ds(0, RG), pl.ds(wbase, PW)], xb0, sem0)
    pltpu.async_copy(x_hbm.at[pl.ds(RG, RG), pl.ds(wbase, PW)], xb1, sem1)

    pltpu.sync_copy(seg_hbm.at[pl.ds(wbase, PW)], segv)
    lane = lax.iota(jnp.int32, L)

    def ib(i, c):
        idxp[pl.ds(i * L, L)] = segv[pl.ds(i * L, L)] * L + lane
        return c

    lax.fori_loop(0, PW // L, ib, 0)

    zv = jnp.zeros((L,), jnp.float32)

    def zb(n, c):
        for j in range(SEGW // L):
            acc2[n, pl.ds(j * L, L)] = zv
        return c

    lax.fori_loop(0, B, zb, 0)

    bufs = (xb0, xb1)
    sems = (sem0, sem1)

    def start(rg, b):
        pltpu.async_copy(
            x_hbm.at[pl.ds(rg * RG, RG), pl.ds(wbase, PW)],
            bufs[b], sems[b])

    def wait(b):
        pltpu.make_async_copy(
            x_hbm.at[pl.ds(0, RG), pl.ds(0, PW)], bufs[b], sems[b]).wait()

    def compute(rg, b):
        buf = bufs[b]
        rows = [jnp.full((L,), rg * RG + r, jnp.int32) for r in range(RG)]

        @plsc.parallel_loop(0, PW // L, unroll=2)
        def vb(v):
            iv = idxp[pl.ds(v * L, L)]
            xs = [buf[r, pl.ds(v * L, L)] for r in range(RG)]
            for r in range(RG):
                plsc.addupdate_scatter(acc2, [rows[r], iv], xs[r])

    def flush(rg):
        pltpu.async_copy(acc2.at[pl.ds(rg * RG, RG), :],
                         out_hbm.at[wid, pl.ds(rg * RG, RG), :], fsem)

    def pair(tp, c):
        for b in range(2):
            t = tp * 2 + b
            wait(b)
            compute(t, b)
            start(t + 2, b)
            flush(t)
        return c

    lax.fori_loop(0, NRG // 2 - 1, pair, 0)
    for b in range(2):
        wait(b)
        compute(NRG - 2 + b, b)
        flush(NRG - 2 + b)
    for _ in range(NRG):
        pltpu.make_async_copy(acc2.at[pl.ds(0, RG), :],
                              out_hbm.at[wid, pl.ds(0, RG), :], fsem).wait()


_sc_call = pl.kernel(
    _sc_body,
    out_type=jax.ShapeDtypeStruct((NW, B, SEGW), jnp.float32),
    mesh=plsc.VectorSubcoreMesh(
        core_axis_name="c", subcore_axis_name="s",
        num_cores=NC, num_subcores=NS),
    compiler_params=pltpu.CompilerParams(needs_layout_passes=False),
    scratch_types=[
        pltpu.VMEM((PW,), jnp.int32),
        pltpu.VMEM((PW,), jnp.int32),
        pltpu.VMEM((B, SEGW), jnp.float32),
        pltpu.VMEM((RG, PW), jnp.float32),
        pltpu.VMEM((RG, PW), jnp.float32),
        pltpu.SemaphoreType.DMA,
        pltpu.SemaphoreType.DMA,
        pltpu.SemaphoreType.DMA,
    ],
)


def _tc_seg_body(x_ref, seg_ref, out_ref):
    i = pl.program_id(0)
    sb = seg_ref[0]
    oh = (sb == lax.broadcasted_iota(jnp.int32, (L, BLKC), 0)
          ).astype(jnp.float32)
    part = lax.dot_general(x_ref[...], oh, (((1,), (1,)), ((), ())),
                           preferred_element_type=jnp.float32)

    @pl.when(i == 0)
    def _():
        out_ref[...] = part

    @pl.when(i > 0)
    def _():
        out_ref[...] += part


_tc_seg_call = pl.pallas_call(
    _tc_seg_body,
    grid=(TCBLKS,),
    in_specs=[
        pl.BlockSpec((B, BLKC), lambda i: (0, TCOFF + i)),
        pl.BlockSpec((1, 1, BLKC), lambda i: (TCOFF + i, 0, 0)),
    ],
    out_specs=pl.BlockSpec((B, L), lambda i: (0, 0)),
    out_shape=jax.ShapeDtypeStruct((B, L), jnp.float32),
)


NFGL = NSEG * COUT


def _ep_body(p_ref, tcp_ref, wpack_ref, wfc_ref, out_ref):
    acc = p_ref[0]
    for w in range(1, NW):
        acc = acc + p_ref[w]
    sel = (lax.broadcasted_iota(jnp.int32, (SEGW, L), 0) // L ==
           lax.broadcasted_iota(jnp.int32, (SEGW, L), 1)).astype(jnp.float32)
    seg16 = (jnp.dot(acc, sel, preferred_element_type=jnp.float32)
             + tcp_ref[...])
    rep = (lax.broadcasted_iota(jnp.int32, (L, NFGL), 1) // COUT ==
           lax.broadcasted_iota(jnp.int32, (L, NFGL), 0)).astype(jnp.float32)
    seg_rep = jnp.dot(seg16, rep, preferred_element_type=jnp.float32)
    wf = jnp.reshape(wpack_ref[pl.ds(0, NFGL)], (1, NFGL))
    bf = jnp.reshape(wpack_ref[pl.ds(NFGL, NFGL)], (1, NFGL))
    bc = jnp.reshape(wpack_ref[pl.ds(2 * NFGL, NCLS)], (1, NCLS))
    flat = seg_rep * wf + bf
    out_ref[...] = (jnp.dot(flat, wfc_ref[...],
                            preferred_element_type=jnp.float32) + bc)


_ep_call = pl.pallas_call(
    _ep_body,
    out_shape=jax.ShapeDtypeStruct((B, NCLS), jnp.float32),
)


def kernel(x, segment_ids, W_fgl, b_fgl, W_fc, b_fc):
    seg32 = segment_ids.astype(jnp.int32)
    part = _sc_call(x, seg32)
    tcp = _tc_seg_call(x, seg32.reshape(P // BLKC, 1, BLKC))
    wpack = jnp.concatenate([W_fgl.reshape(-1), b_fgl.reshape(-1), b_fc])
    return _ep_call(part, tcp, wpack, W_fc)
